# baseline (speedup 1.0000x reference)
import functools

import jax
import jax.numpy as jnp
from jax import lax
from jax.experimental import pallas as pl
from jax.experimental.pallas import tpu as pltpu
from jax.experimental.pallas import tpu_sc as plsc

B = 16384
DIM = 64
N_ENT = 1000000
N_REL = 1000
_TAU = 0.0

_info = plsc.get_sparse_core_info()
_NC = _info.num_cores
_NS = _info.num_subcores
_NW = _NC * _NS
_TCOLS = (N_ENT + 127) // 128
_CPW = 245
_CHC = 1
_NCH = 245
_SB = B + 8
_GRP = 32

_mesh = plsc.VectorSubcoreMesh(core_axis_name="c", subcore_axis_name="s")
_params = pltpu.CompilerParams(needs_layout_passes=False)

_LANE = None


def _splat(x):
    return jnp.zeros((16,), jnp.int32) + x


@functools.partial(
    pl.kernel,
    mesh=_mesh,
    out_type=(
        jax.ShapeDtypeStruct((_SB, 128), jnp.float32),
        jax.ShapeDtypeStruct((_SB, 128), jnp.float32),
        jax.ShapeDtypeStruct((_SB, 128), jnp.float32),
        jax.ShapeDtypeStruct((_SB, 128), jnp.float32),
    ),
    compiler_params=_params,
    scratch_types=[
        pltpu.VMEM((2, 64, 128 * _CHC), jnp.float32),
        pltpu.VMEM((2, 64, 128 * _CHC), jnp.float32),
        pltpu.VMEM((2048,), jnp.int32),
        pltpu.VMEM((2048,), jnp.int32),
        pltpu.VMEM((B + 160,), jnp.int32),
        pltpu.VMEM((B + 160,), jnp.int32),
        pltpu.VMEM((B + 16,), jnp.int32),
        pltpu.VMEM((2, _GRP, 128), jnp.float32),
        pltpu.VMEM((2, _GRP, 128), jnp.float32),
        pltpu.VMEM((2, _GRP, 128), jnp.float32),
        pltpu.VMEM((2, _GRP, 128), jnp.float32),
        pltpu.VMEM((2, _GRP), jnp.int32),
        pltpu.VMEM((2, _GRP), jnp.int32),
        pltpu.SemaphoreType.DMA,
        pltpu.SemaphoreType.DMA,
        pltpu.SemaphoreType.DMA,
        pltpu.SemaphoreType.DMA,
        pltpu.SemaphoreType.DMA,
        pltpu.SemaphoreType.DMA,
    ],
)
def _sweep_kernel(entT_re, entT_im, heads, tails,
                  hre, him, tre, tim,
                  chk_re, chk_im, hblk, tblk, hlist, tlist, tmpl,
                  hrow_re, hrow_im, trow_re, trow_im, hbidx, tbidx,
                  csem0, csem1, hsem0, hsem1, tsem0, tsem1):
    lane = lax.iota(jnp.int32, 16)
    wid = lax.axis_index("s") * _NC + lax.axis_index("c")
    wcol0 = wid * _CPW
    wlo = wcol0 * 128
    whi = jnp.minimum(wlo + _CPW * 128, N_ENT)

    rows_ref = {"h": (hrow_re, hrow_im), "t": (trow_re, trow_im)}
    bidx_ref = {"h": hbidx, "t": tbidx}
    outs_ref = {"h": (hre, him), "t": (tre, tim)}
    sems = {"h": (hsem0, hsem1), "t": (tsem0, tsem1)}

    for side in ("h", "t"):
        for p in (0, 1):
            for q in range(_GRP // 16):
                bidx_ref[side][p, pl.ds(q * 16, 16)] = _splat(B)

    zero = jnp.zeros((), jnp.int32)
    _NSUB = 8

    def build_side(src, blk, lst):
        def bblk(blk_i, cnt):
            pltpu.sync_copy(src.at[pl.ds(blk_i * 2048, 2048)], blk)

            def sg(g, c):
                ids = blk[pl.ds(g * 16, 16)]
                b_vec = blk_i * 2048 + g * 16 + lane
                m = (ids >= wlo) & (ids < whi)
                plsc.store_compressed(tmpl.at[pl.ds(c, 16)],
                                      (ids - wlo) * 16384 + b_vec, mask=m)
                return c + plsc.all_reduce_population_count(m)[0]

            return lax.fori_loop(0, 128, sg, cnt)

        cnt = lax.fori_loop(0, 8, bblk, zero)
        ngrp = (cnt + 15) // 16

        def hist(g, cs):
            vals = tmpl[pl.ds(g * 16, 16)]
            mt = lane < (cnt - g * 16)
            bk = vals >> 26
            return tuple(
                cs[bb] + plsc.all_reduce_population_count(
                    mt & (bk == bb))[0] for bb in range(_NSUB))

        cs = lax.fori_loop(0, ngrp, hist, (zero,) * _NSUB)
        offs = [zero]
        for bb in range(_NSUB):
            offs.append((offs[bb] + cs[bb] + 15) // 16 * 16)

        def fill(i, c):
            lst[pl.ds(i * 16, 16)] = _splat(0x7FFFFFFF)
            return c

        lax.fori_loop(0, (B + 160) // 16, fill, zero)

        def place(g, ptrs):
            vals = tmpl[pl.ds(g * 16, 16)]
            mt = lane < (cnt - g * 16)
            bk = vals >> 26
            nptrs = []
            for bb in range(_NSUB):
                mb = mt & (bk == bb)
                plsc.store_compressed(lst.at[pl.ds(ptrs[bb], 16)], vals,
                                      mask=mb)
                nptrs.append(
                    ptrs[bb] + plsc.all_reduce_population_count(mb)[0])
            return tuple(nptrs)

        lax.fori_loop(0, ngrp, place, tuple(offs[:_NSUB]))
        return offs[:_NSUB], cs

    hoffs, hcs = build_side(heads, hblk, hlist)
    toffs, tcs = build_side(tails, tblk, tlist)

    cvecs = [lax.iota(jnp.int32, 16) + c0 * 16 for c0 in range(4)]

    _W = 128 * _CHC

    def col_start(j):
        return jnp.minimum(wcol0 + j * _CHC, _TCOLS - _CHC) * 128

    def issue(j, sem):
        s0 = col_start(j)
        cp1 = pltpu.async_copy(
            entT_re.at[:, pl.ds(s0, _W)], chk_re.at[j % 2], sem)
        cp2 = pltpu.async_copy(
            entT_im.at[:, pl.ds(s0, _W)], chk_im.at[j % 2], sem)
        return cp1, cp2

    def drain_chunk(sem):
        pltpu.make_async_copy(
            entT_re.at[:, pl.ds(0, _W)], chk_re.at[0], sem).wait()
        pltpu.make_async_copy(
            entT_im.at[:, pl.ds(0, _W)], chk_im.at[0], sem).wait()

    def fire(side, p):
        rre, rim = rows_ref[side]
        ore, oim = outs_ref[side]
        sem = sems[side][p]
        idx = bidx_ref[side].at[p]
        pltpu.async_copy(rre.at[p], ore.at[idx], sem)
        pltpu.async_copy(rim.at[p], oim.at[idx], sem)

    def drain_scatter(side, p):
        rre, rim = rows_ref[side]
        sem = sems[side][p]
        pltpu.make_async_copy(
            entT_re.at[pl.ds(0, _GRP), pl.ds(0, 128)], rre.at[p], sem).wait()
        pltpu.make_async_copy(
            entT_re.at[pl.ds(0, _GRP), pl.ds(0, 128)], rim.at[p], sem).wait()

    issue(0, csem0)

    def chunk_body(jj, carry, bb=0):
        j = bb * 32 + jj
        kh, ph, kt, pt, oh0, oh1, ot0, ot1 = carry
        jp = j % 2
        col = wcol0 + j * _CHC
        valid = (j < _NCH) & (col < _TCOLS)
        nxt = (j + 1 < _NCH) & (col + _CHC < _TCOLS)

        @pl.when(nxt & (jp == 0))
        def _():
            issue(j + 1, csem1)

        @pl.when(nxt & (jp == 1))
        def _():
            issue(j + 1, csem0)

        @pl.when(valid & (jp == 0))
        def _():
            drain_chunk(csem0)

        @pl.when(valid & (jp == 1))
        def _():
            drain_chunk(csem1)

        jp_s = _splat(jp)
        s0 = col_start(j)
        blo = (col * 128 - wlo) * 16384
        bhi = blo + (128 * _CHC) * 16384

        def side_scan(side, base, cnt_bb, scar):
            lst = hlist if side == "h" else tlist
            ngrp = (cnt_bb + 15) // 16

            def extract_one(vals, st):
                m_, k, p, o0, o1 = st
                li = plsc.all_reduce_ffs(m_)
                v = vals.at[li].get(mode="promise_in_bounds")
                m2 = m_ & (lane != li)
                b_s = v & 16383
                lid = v >> 14
                l_s = lid + wlo - s0
                rre, rim = rows_ref[side]
                for c0 in range(4):
                    gre = plsc.load_gather(chk_re, [jp_s, cvecs[c0], l_s])
                    gim = plsc.load_gather(chk_im, [jp_s, cvecs[c0], l_s])
                    rre[p, k, pl.ds(c0 * 16, 16)] = gre
                    rim[p, k, pl.ds(c0 * 16, 16)] = gim
                gg = k // 16
                bref = bidx_ref[side]
                cur = bref[p, pl.ds(gg * 16, 16)]
                bref[p, pl.ds(gg * 16, 16)] = jnp.where(
                    lane == (k - gg * 16), b_s, cur)
                k = k + 1
                full = k >= _GRP

                @pl.when(full & (p == 0))
                def _():
                    fire(side, 0)

                    @pl.when(o1 > 0)
                    def _():
                        drain_scatter(side, 1)

                @pl.when(full & (p == 1))
                def _():
                    fire(side, 1)

                    @pl.when(o0 > 0)
                    def _():
                        drain_scatter(side, 0)

                o0 = jnp.where(full & (p == 0), 1, jnp.where(full, 0, o0))
                o1 = jnp.where(full & (p == 1), 1, jnp.where(full, 0, o1))
                p = jnp.where(full, 1 - p, p)
                k = jnp.where(full, 0, k)
                return m2, k, p, o0, o1

            def grp_body(g, c2):
                vals = lst[pl.ds(base + g * 16, 16)]
                m0 = (vals >= blo) & (vals < bhi)
                res = lax.while_loop(
                    lambda s: jnp.any(s[0]),
                    lambda s, _v=vals: extract_one(_v, s),
                    (m0,) + c2)
                return res[1:]

            return lax.fori_loop(0, ngrp, grp_body, scar)

        kh, ph, oh0, oh1 = side_scan(
            "h", hoffs[bb], hcs[bb], (kh, ph, oh0, oh1))
        kt, pt, ot0, ot1 = side_scan(
            "t", toffs[bb], tcs[bb], (kt, pt, ot0, ot1))
        return kh, ph, kt, pt, oh0, oh1, ot0, ot1

    carry = (zero, zero, zero, zero, zero, zero, zero, zero)
    for _bb in range(_NSUB):
        carry = lax.fori_loop(
            0, 32, functools.partial(chunk_body, bb=_bb), carry)
    kh, ph, kt, pt, oh0, oh1, ot0, ot1 = carry

    @pl.when((kh > 0) & (ph == 0))
    def _():
        fire("h", 0)

    @pl.when((kh > 0) & (ph == 1))
    def _():
        fire("h", 1)

    @pl.when((kt > 0) & (pt == 0))
    def _():
        fire("t", 0)

    @pl.when((kt > 0) & (pt == 1))
    def _():
        fire("t", 1)

    oh0 = jnp.where((kh > 0) & (ph == 0), 1, oh0)
    oh1 = jnp.where((kh > 0) & (ph == 1), 1, oh1)
    ot0 = jnp.where((kt > 0) & (pt == 0), 1, ot0)
    ot1 = jnp.where((kt > 0) & (pt == 1), 1, ot1)

    @pl.when(oh0 > 0)
    def _():
        drain_scatter("h", 0)

    @pl.when(oh1 > 0)
    def _():
        drain_scatter("h", 1)

    @pl.when(ot0 > 0)
    def _():
        drain_scatter("t", 0)

    @pl.when(ot1 > 0)
    def _():
        drain_scatter("t", 1)


@functools.partial(
    pl.kernel,
    mesh=_mesh,
    out_type=jax.ShapeDtypeStruct((B,), jnp.float32),
    compiler_params=_params,
    scratch_types=[
        pltpu.VMEM((64, 128), jnp.float32),
        pltpu.VMEM((64, 128), jnp.float32),
        pltpu.VMEM((64, 128), jnp.float32),
        pltpu.VMEM((64, 128), jnp.float32),
        pltpu.VMEM((64, 128), jnp.float32),
        pltpu.VMEM((64, 128), jnp.float32),
        pltpu.VMEM((64,), jnp.int32),
        pltpu.VMEM((64,), jnp.float32),
        pltpu.SemaphoreType.DMA,
    ],
)
def _score_kernel(hre, him, tre, tim, rel_re, rel_im, rels, out,
                  bh_re, bh_im, bt_re, bt_im, brr, bri, ridx, outv, sem):
    lane = lax.iota(jnp.int32, 16)
    wid = lax.axis_index("s") * _NC + lax.axis_index("c")
    wb = wid * (B // _NW)

    def sub_body(sc, carry):
        base = wb + sc * 64
        pltpu.sync_copy(rels.at[pl.ds(base, 64)], ridx)
        cps = [
            pltpu.async_copy(hre.at[pl.ds(base, 64), :], bh_re, sem),
            pltpu.async_copy(him.at[pl.ds(base, 64), :], bh_im, sem),
            pltpu.async_copy(tre.at[pl.ds(base, 64), :], bt_re, sem),
            pltpu.async_copy(tim.at[pl.ds(base, 64), :], bt_im, sem),
            pltpu.async_copy(rel_re.at[ridx], brr, sem),
            pltpu.async_copy(rel_im.at[ridx], bri, sem),
        ]
        for cp in cps:
            cp.wait()

        def group(g, c2):
            out16 = jnp.zeros((16,), jnp.float32)
            for jj in range(16):
                row = g * 16 + jj
                acc = None
                for c0 in range(4):
                    sl = pl.ds(c0 * 16, 16)
                    hr = bh_re[row, sl]
                    hi = bh_im[row, sl]
                    tr = bt_re[row, sl]
                    ti = bt_im[row, sl]
                    rr = brr[row, sl]
                    ri = bri[row, sl]
                    term = rr * (hr * tr + hi * ti) + ri * (hr * ti - hi * tr)
                    acc = term if acc is None else acc + term
                s = lax.reduce_sum(acc, axes=(0,))
                out16 = jnp.where(lane == jj, s, out16)
            outv[pl.ds(g * 16, 16)] = out16
            return c2

        lax.fori_loop(0, 4, group, 0)
        pltpu.sync_copy(outv, out.at[pl.ds(base, 64)])
        return carry

    lax.fori_loop(0, 8, sub_body, 0)


def kernel(heads, rels, tails, ent_re, ent_im, rel_re, rel_im):
    heads = heads.astype(jnp.int32)
    rels = rels.astype(jnp.int32)
    tails = tails.astype(jnp.int32)
    hre, him, tre, tim = _sweep_kernel(ent_re.T, ent_im.T, heads, tails)
    rel_re128 = jnp.pad(rel_re, ((0, 0), (0, 64)))
    rel_im128 = jnp.pad(rel_im, ((0, 0), (0, 64)))
    score = _score_kernel(hre, him, tre, tim, rel_re128, rel_im128, rels)
    return score - _TAU


# speedup vs baseline: 1.0174x; 1.0174x over previous
import functools

import jax
import jax.numpy as jnp
from jax import lax
from jax.experimental import pallas as pl
from jax.experimental.pallas import tpu as pltpu
from jax.experimental.pallas import tpu_sc as plsc

B = 16384
DIM = 64
N_ENT = 1000000
N_REL = 1000
_TAU = 0.0

_info = plsc.get_sparse_core_info()
_NC = _info.num_cores
_NS = _info.num_subcores
_NW = _NC * _NS
_TCOLS = (N_ENT + 127) // 128
_CPW = 245
_CHC = 1
_NCH = 245
_SB = B + 8
_GRP = 32

_mesh = plsc.VectorSubcoreMesh(core_axis_name="c", subcore_axis_name="s")
_params = pltpu.CompilerParams(needs_layout_passes=False)

_LANE = None


def _splat(x):
    return jnp.zeros((16,), jnp.int32) + x


@functools.partial(
    pl.kernel,
    mesh=_mesh,
    out_type=(
        jax.ShapeDtypeStruct((_SB, 128), jnp.float32),
        jax.ShapeDtypeStruct((_SB, 128), jnp.float32),
        jax.ShapeDtypeStruct((_SB, 128), jnp.float32),
        jax.ShapeDtypeStruct((_SB, 128), jnp.float32),
    ),
    compiler_params=_params,
    scratch_types=[
        pltpu.VMEM((2, 64, 128 * _CHC), jnp.float32),
        pltpu.VMEM((2, 64, 128 * _CHC), jnp.float32),
        pltpu.VMEM((2048,), jnp.int32),
        pltpu.VMEM((2048,), jnp.int32),
        pltpu.VMEM((B + 272,), jnp.int32),
        pltpu.VMEM((B + 272,), jnp.int32),
        pltpu.VMEM((B + 16,), jnp.int32),
        pltpu.VMEM((2, _GRP, 128), jnp.float32),
        pltpu.VMEM((2, _GRP, 128), jnp.float32),
        pltpu.VMEM((2, _GRP, 128), jnp.float32),
        pltpu.VMEM((2, _GRP, 128), jnp.float32),
        pltpu.VMEM((2, _GRP), jnp.int32),
        pltpu.VMEM((2, _GRP), jnp.int32),
        pltpu.SemaphoreType.DMA,
        pltpu.SemaphoreType.DMA,
        pltpu.SemaphoreType.DMA,
        pltpu.SemaphoreType.DMA,
        pltpu.SemaphoreType.DMA,
        pltpu.SemaphoreType.DMA,
    ],
)
def _sweep_kernel(entT_re, entT_im, heads, tails,
                  hre, him, tre, tim,
                  chk_re, chk_im, hblk, tblk, hlist, tlist, tmpl,
                  hrow_re, hrow_im, trow_re, trow_im, hbidx, tbidx,
                  csem0, csem1, hsem0, hsem1, tsem0, tsem1):
    lane = lax.iota(jnp.int32, 16)
    wid = lax.axis_index("s") * _NC + lax.axis_index("c")
    wcol0 = wid * _CPW
    wlo = wcol0 * 128
    whi = jnp.minimum(wlo + _CPW * 128, N_ENT)

    rows_ref = {"h": (hrow_re, hrow_im), "t": (trow_re, trow_im)}
    bidx_ref = {"h": hbidx, "t": tbidx}
    outs_ref = {"h": (hre, him), "t": (tre, tim)}
    sems = {"h": (hsem0, hsem1), "t": (tsem0, tsem1)}

    for side in ("h", "t"):
        for p in (0, 1):
            for q in range(_GRP // 16):
                bidx_ref[side][p, pl.ds(q * 16, 16)] = _splat(B)

    zero = jnp.zeros((), jnp.int32)
    _NSUB = 16

    def build_side(src, blk, lst):
        def bblk(blk_i, cnt):
            pltpu.sync_copy(src.at[pl.ds(blk_i * 2048, 2048)], blk)

            def sg(g, c):
                ids = blk[pl.ds(g * 16, 16)]
                b_vec = blk_i * 2048 + g * 16 + lane
                m = (ids >= wlo) & (ids < whi)
                plsc.store_compressed(tmpl.at[pl.ds(c, 16)],
                                      (ids - wlo) * 16384 + b_vec, mask=m)
                return c + plsc.all_reduce_population_count(m)[0]

            return lax.fori_loop(0, 128, sg, cnt)

        cnt = lax.fori_loop(0, 8, bblk, zero)
        ngrp = (cnt + 15) // 16

        def hist(g, cs):
            vals = tmpl[pl.ds(g * 16, 16)]
            mt = lane < (cnt - g * 16)
            bk = vals >> 25
            return tuple(
                cs[bb] + plsc.all_reduce_population_count(
                    mt & (bk == bb))[0] for bb in range(_NSUB))

        cs = lax.fori_loop(0, ngrp, hist, (zero,) * _NSUB)
        offs = [zero]
        for bb in range(_NSUB):
            offs.append((offs[bb] + cs[bb] + 15) // 16 * 16)

        def fill(i, c):
            lst[pl.ds(i * 16, 16)] = _splat(0x7FFFFFFF)
            return c

        lax.fori_loop(0, (B + 272) // 16, fill, zero)

        def place(g, ptrs):
            vals = tmpl[pl.ds(g * 16, 16)]
            mt = lane < (cnt - g * 16)
            bk = vals >> 25
            nptrs = []
            for bb in range(_NSUB):
                mb = mt & (bk == bb)
                plsc.store_compressed(lst.at[pl.ds(ptrs[bb], 16)], vals,
                                      mask=mb)
                nptrs.append(
                    ptrs[bb] + plsc.all_reduce_population_count(mb)[0])
            return tuple(nptrs)

        lax.fori_loop(0, ngrp, place, tuple(offs[:_NSUB]))
        return offs[:_NSUB], cs

    hoffs, hcs = build_side(heads, hblk, hlist)
    toffs, tcs = build_side(tails, tblk, tlist)

    cvecs = [lax.iota(jnp.int32, 16) + c0 * 16 for c0 in range(4)]

    _W = 128 * _CHC

    def col_start(j):
        return jnp.minimum(wcol0 + j * _CHC, _TCOLS - _CHC) * 128

    def issue(j, sem):
        s0 = col_start(j)
        cp1 = pltpu.async_copy(
            entT_re.at[:, pl.ds(s0, _W)], chk_re.at[j % 2], sem)
        cp2 = pltpu.async_copy(
            entT_im.at[:, pl.ds(s0, _W)], chk_im.at[j % 2], sem)
        return cp1, cp2

    def drain_chunk(sem):
        pltpu.make_async_copy(
            entT_re.at[:, pl.ds(0, _W)], chk_re.at[0], sem).wait()
        pltpu.make_async_copy(
            entT_im.at[:, pl.ds(0, _W)], chk_im.at[0], sem).wait()

    def fire(side, p):
        rre, rim = rows_ref[side]
        ore, oim = outs_ref[side]
        sem = sems[side][p]
        idx = bidx_ref[side].at[p]
        pltpu.async_copy(rre.at[p], ore.at[idx], sem)
        pltpu.async_copy(rim.at[p], oim.at[idx], sem)

    def drain_scatter(side, p):
        rre, rim = rows_ref[side]
        sem = sems[side][p]
        pltpu.make_async_copy(
            entT_re.at[pl.ds(0, _GRP), pl.ds(0, 128)], rre.at[p], sem).wait()
        pltpu.make_async_copy(
            entT_re.at[pl.ds(0, _GRP), pl.ds(0, 128)], rim.at[p], sem).wait()

    issue(0, csem0)

    def chunk_body(jj, carry, bb=0):
        j = bb * 16 + jj
        kh, ph, kt, pt, oh0, oh1, ot0, ot1 = carry
        jp = j % 2
        col = wcol0 + j * _CHC
        valid = (j < _NCH) & (col < _TCOLS)
        nxt = (j + 1 < _NCH) & (col + _CHC < _TCOLS)

        @pl.when(nxt & (jp == 0))
        def _():
            issue(j + 1, csem1)

        @pl.when(nxt & (jp == 1))
        def _():
            issue(j + 1, csem0)

        @pl.when(valid & (jp == 0))
        def _():
            drain_chunk(csem0)

        @pl.when(valid & (jp == 1))
        def _():
            drain_chunk(csem1)

        jp_s = _splat(jp)
        s0 = col_start(j)
        blo = (col * 128 - wlo) * 16384
        bhi = blo + (128 * _CHC) * 16384

        def side_scan(side, base, cnt_bb, scar):
            lst = hlist if side == "h" else tlist
            ngrp = (cnt_bb + 15) // 16

            def extract_one(vals, st):
                m_, k, p, o0, o1 = st
                li = plsc.all_reduce_ffs(m_)
                v = vals.at[li].get(mode="promise_in_bounds")
                m2 = m_ & (lane != li)
                b_s = v & 16383
                lid = v >> 14
                l_s = lid + wlo - s0
                rre, rim = rows_ref[side]
                for c0 in range(4):
                    gre = plsc.load_gather(chk_re, [jp_s, cvecs[c0], l_s])
                    gim = plsc.load_gather(chk_im, [jp_s, cvecs[c0], l_s])
                    rre[p, k, pl.ds(c0 * 16, 16)] = gre
                    rim[p, k, pl.ds(c0 * 16, 16)] = gim
                gg = k // 16
                bref = bidx_ref[side]
                cur = bref[p, pl.ds(gg * 16, 16)]
                bref[p, pl.ds(gg * 16, 16)] = jnp.where(
                    lane == (k - gg * 16), b_s, cur)
                k = k + 1
                full = k >= _GRP

                @pl.when(full & (p == 0))
                def _():
                    fire(side, 0)

                    @pl.when(o1 > 0)
                    def _():
                        drain_scatter(side, 1)

                @pl.when(full & (p == 1))
                def _():
                    fire(side, 1)

                    @pl.when(o0 > 0)
                    def _():
                        drain_scatter(side, 0)

                o0 = jnp.where(full & (p == 0), 1, jnp.where(full, 0, o0))
                o1 = jnp.where(full & (p == 1), 1, jnp.where(full, 0, o1))
                p = jnp.where(full, 1 - p, p)
                k = jnp.where(full, 0, k)
                return m2, k, p, o0, o1

            def grp_body(g, c2):
                vals = lst[pl.ds(base + g * 16, 16)]
                m0 = (vals >= blo) & (vals < bhi)
                res = lax.while_loop(
                    lambda s: jnp.any(s[0]),
                    lambda s, _v=vals: extract_one(_v, s),
                    (m0,) + c2)
                return res[1:]

            return lax.fori_loop(0, ngrp, grp_body, scar)

        kh, ph, oh0, oh1 = side_scan(
            "h", hoffs[bb], hcs[bb], (kh, ph, oh0, oh1))
        kt, pt, ot0, ot1 = side_scan(
            "t", toffs[bb], tcs[bb], (kt, pt, ot0, ot1))
        return kh, ph, kt, pt, oh0, oh1, ot0, ot1

    carry = (zero, zero, zero, zero, zero, zero, zero, zero)
    for _bb in range(_NSUB):
        carry = lax.fori_loop(
            0, 16, functools.partial(chunk_body, bb=_bb), carry)
    kh, ph, kt, pt, oh0, oh1, ot0, ot1 = carry

    @pl.when((kh > 0) & (ph == 0))
    def _():
        fire("h", 0)

    @pl.when((kh > 0) & (ph == 1))
    def _():
        fire("h", 1)

    @pl.when((kt > 0) & (pt == 0))
    def _():
        fire("t", 0)

    @pl.when((kt > 0) & (pt == 1))
    def _():
        fire("t", 1)

    oh0 = jnp.where((kh > 0) & (ph == 0), 1, oh0)
    oh1 = jnp.where((kh > 0) & (ph == 1), 1, oh1)
    ot0 = jnp.where((kt > 0) & (pt == 0), 1, ot0)
    ot1 = jnp.where((kt > 0) & (pt == 1), 1, ot1)

    @pl.when(oh0 > 0)
    def _():
        drain_scatter("h", 0)

    @pl.when(oh1 > 0)
    def _():
        drain_scatter("h", 1)

    @pl.when(ot0 > 0)
    def _():
        drain_scatter("t", 0)

    @pl.when(ot1 > 0)
    def _():
        drain_scatter("t", 1)


@functools.partial(
    pl.kernel,
    mesh=_mesh,
    out_type=jax.ShapeDtypeStruct((B,), jnp.float32),
    compiler_params=_params,
    scratch_types=[
        pltpu.VMEM((64, 128), jnp.float32),
        pltpu.VMEM((64, 128), jnp.float32),
        pltpu.VMEM((64, 128), jnp.float32),
        pltpu.VMEM((64, 128), jnp.float32),
        pltpu.VMEM((64, 128), jnp.float32),
        pltpu.VMEM((64, 128), jnp.float32),
        pltpu.VMEM((64,), jnp.int32),
        pltpu.VMEM((64,), jnp.float32),
        pltpu.SemaphoreType.DMA,
    ],
)
def _score_kernel(hre, him, tre, tim, rel_re, rel_im, rels, out,
                  bh_re, bh_im, bt_re, bt_im, brr, bri, ridx, outv, sem):
    lane = lax.iota(jnp.int32, 16)
    wid = lax.axis_index("s") * _NC + lax.axis_index("c")
    wb = wid * (B // _NW)

    def sub_body(sc, carry):
        base = wb + sc * 64
        pltpu.sync_copy(rels.at[pl.ds(base, 64)], ridx)
        cps = [
            pltpu.async_copy(hre.at[pl.ds(base, 64), :], bh_re, sem),
            pltpu.async_copy(him.at[pl.ds(base, 64), :], bh_im, sem),
            pltpu.async_copy(tre.at[pl.ds(base, 64), :], bt_re, sem),
            pltpu.async_copy(tim.at[pl.ds(base, 64), :], bt_im, sem),
            pltpu.async_copy(rel_re.at[ridx], brr, sem),
            pltpu.async_copy(rel_im.at[ridx], bri, sem),
        ]
        for cp in cps:
            cp.wait()

        def group(g, c2):
            out16 = jnp.zeros((16,), jnp.float32)
            for jj in range(16):
                row = g * 16 + jj
                acc = None
                for c0 in range(4):
                    sl = pl.ds(c0 * 16, 16)
                    hr = bh_re[row, sl]
                    hi = bh_im[row, sl]
                    tr = bt_re[row, sl]
                    ti = bt_im[row, sl]
                    rr = brr[row, sl]
                    ri = bri[row, sl]
                    term = rr * (hr * tr + hi * ti) + ri * (hr * ti - hi * tr)
                    acc = term if acc is None else acc + term
                s = lax.reduce_sum(acc, axes=(0,))
                out16 = jnp.where(lane == jj, s, out16)
            outv[pl.ds(g * 16, 16)] = out16
            return c2

        lax.fori_loop(0, 4, group, 0)
        pltpu.sync_copy(outv, out.at[pl.ds(base, 64)])
        return carry

    lax.fori_loop(0, 8, sub_body, 0)


def kernel(heads, rels, tails, ent_re, ent_im, rel_re, rel_im):
    heads = heads.astype(jnp.int32)
    rels = rels.astype(jnp.int32)
    tails = tails.astype(jnp.int32)
    hre, him, tre, tim = _sweep_kernel(ent_re.T, ent_im.T, heads, tails)
    rel_re128 = jnp.pad(rel_re, ((0, 0), (0, 64)))
    rel_im128 = jnp.pad(rel_im, ((0, 0), (0, 64)))
    score = _score_kernel(hre, him, tre, tim, rel_re128, rel_im128, rels)
    return score - _TAU


# speedup vs baseline: 1.0545x; 1.0365x over previous
import functools

import jax
import jax.numpy as jnp
from jax import lax
from jax.experimental import pallas as pl
from jax.experimental.pallas import tpu as pltpu
from jax.experimental.pallas import tpu_sc as plsc

B = 16384
DIM = 64
N_ENT = 1000000
N_REL = 1000
_TAU = 0.0

_info = plsc.get_sparse_core_info()
_NC = _info.num_cores
_NS = _info.num_subcores
_NW = _NC * _NS
_TCOLS = (N_ENT + 127) // 128
_CPW = 245
_CHC = 1
_NCH = 245
_SB = B + 8
_GRP = 32

_mesh = plsc.VectorSubcoreMesh(core_axis_name="c", subcore_axis_name="s")
_params = pltpu.CompilerParams(needs_layout_passes=False)

_LANE = None


def _splat(x):
    return jnp.zeros((16,), jnp.int32) + x


@functools.partial(
    pl.kernel,
    mesh=_mesh,
    out_type=(
        jax.ShapeDtypeStruct((_SB, 128), jnp.float32),
        jax.ShapeDtypeStruct((_SB, 128), jnp.float32),
        jax.ShapeDtypeStruct((_SB, 128), jnp.float32),
        jax.ShapeDtypeStruct((_SB, 128), jnp.float32),
    ),
    compiler_params=_params,
    scratch_types=[
        pltpu.VMEM((2, 64, 128 * _CHC), jnp.float32),
        pltpu.VMEM((2, 64, 128 * _CHC), jnp.float32),
        pltpu.VMEM((2048,), jnp.int32),
        pltpu.VMEM((2048,), jnp.int32),
        pltpu.VMEM((B + 272,), jnp.int32),
        pltpu.VMEM((B + 272,), jnp.int32),
        pltpu.VMEM((B + 16,), jnp.int32),
        pltpu.VMEM((2, _GRP, 128), jnp.float32),
        pltpu.VMEM((2, _GRP, 128), jnp.float32),
        pltpu.VMEM((2, _GRP, 128), jnp.float32),
        pltpu.VMEM((2, _GRP, 128), jnp.float32),
        pltpu.VMEM((2, _GRP), jnp.int32),
        pltpu.VMEM((2, _GRP), jnp.int32),
        pltpu.SemaphoreType.DMA,
        pltpu.SemaphoreType.DMA,
        pltpu.SemaphoreType.DMA,
        pltpu.SemaphoreType.DMA,
        pltpu.SemaphoreType.DMA,
        pltpu.SemaphoreType.DMA,
    ],
)
def _sweep_kernel(entT_re, entT_im, heads, tails,
                  hre, him, tre, tim,
                  chk_re, chk_im, hblk, tblk, hlist, tlist, tmpl,
                  hrow_re, hrow_im, trow_re, trow_im, hbidx, tbidx,
                  csem0, csem1, hsem0, hsem1, tsem0, tsem1):
    lane = lax.iota(jnp.int32, 16)
    wid = lax.axis_index("s") * _NC + lax.axis_index("c")
    wcol0 = wid * _CPW
    wlo = wcol0 * 128
    whi = jnp.minimum(wlo + _CPW * 128, N_ENT)

    rows_ref = {"h": (hrow_re, hrow_im), "t": (trow_re, trow_im)}
    bidx_ref = {"h": hbidx, "t": tbidx}
    outs_ref = {"h": (hre, him), "t": (tre, tim)}
    sems = {"h": (hsem0, hsem1), "t": (tsem0, tsem1)}

    for side in ("h", "t"):
        for p in (0, 1):
            for q in range(_GRP // 16):
                bidx_ref[side][p, pl.ds(q * 16, 16)] = _splat(B)

    zero = jnp.zeros((), jnp.int32)
    _NSUB = 16

    def build_side(src, blk, lst):
        def bblk(blk_i, cnt):
            pltpu.sync_copy(src.at[pl.ds(blk_i * 2048, 2048)], blk)

            def sg(g, c):
                ids = blk[pl.ds(g * 16, 16)]
                b_vec = blk_i * 2048 + g * 16 + lane
                m = (ids >= wlo) & (ids < whi)
                plsc.store_compressed(tmpl.at[pl.ds(c, 16)],
                                      (ids - wlo) * 16384 + b_vec, mask=m)
                return c + plsc.all_reduce_population_count(m)[0]

            return lax.fori_loop(0, 128, sg, cnt)

        cnt = lax.fori_loop(0, 8, bblk, zero)
        ngrp = (cnt + 15) // 16

        def hist(g, cs):
            vals = tmpl[pl.ds(g * 16, 16)]
            mt = lane < (cnt - g * 16)
            bk = vals >> 25
            return tuple(
                cs[bb] + plsc.all_reduce_population_count(
                    mt & (bk == bb))[0] for bb in range(_NSUB))

        cs = lax.fori_loop(0, ngrp, hist, (zero,) * _NSUB)
        offs = [zero]
        for bb in range(_NSUB):
            offs.append((offs[bb] + cs[bb] + 15) // 16 * 16)

        def fill(i, c):
            lst[pl.ds(i * 16, 16)] = _splat(0x7FFFFFFF)
            return c

        lax.fori_loop(0, (B + 272) // 16, fill, zero)

        def place(g, ptrs):
            vals = tmpl[pl.ds(g * 16, 16)]
            mt = lane < (cnt - g * 16)
            bk = vals >> 25
            nptrs = []
            for bb in range(_NSUB):
                mb = mt & (bk == bb)
                plsc.store_compressed(lst.at[pl.ds(ptrs[bb], 16)], vals,
                                      mask=mb)
                nptrs.append(
                    ptrs[bb] + plsc.all_reduce_population_count(mb)[0])
            return tuple(nptrs)

        lax.fori_loop(0, ngrp, place, tuple(offs[:_NSUB]))
        return offs[:_NSUB], cs

    hoffs, hcs = build_side(heads, hblk, hlist)
    toffs, tcs = build_side(tails, tblk, tlist)

    cvecs = [lax.iota(jnp.int32, 16) + c0 * 16 for c0 in range(4)]

    _W = 128 * _CHC

    def col_start(j):
        return jnp.minimum(wcol0 + j * _CHC, _TCOLS - _CHC) * 128

    def issue(j, sem):
        s0 = col_start(j)
        cp1 = pltpu.async_copy(
            entT_re.at[:, pl.ds(s0, _W)], chk_re.at[j % 2], sem)
        cp2 = pltpu.async_copy(
            entT_im.at[:, pl.ds(s0, _W)], chk_im.at[j % 2], sem)
        return cp1, cp2

    def drain_chunk(sem):
        pltpu.make_async_copy(
            entT_re.at[:, pl.ds(0, _W)], chk_re.at[0], sem).wait()
        pltpu.make_async_copy(
            entT_im.at[:, pl.ds(0, _W)], chk_im.at[0], sem).wait()

    def fire(side, p):
        rre, rim = rows_ref[side]
        ore, oim = outs_ref[side]
        sem = sems[side][p]
        idx = bidx_ref[side].at[p]
        pltpu.async_copy(rre.at[p], ore.at[idx], sem)
        pltpu.async_copy(rim.at[p], oim.at[idx], sem)

    def drain_scatter(side, p):
        rre, rim = rows_ref[side]
        sem = sems[side][p]
        pltpu.make_async_copy(
            entT_re.at[pl.ds(0, _GRP), pl.ds(0, 128)], rre.at[p], sem).wait()
        pltpu.make_async_copy(
            entT_re.at[pl.ds(0, _GRP), pl.ds(0, 128)], rim.at[p], sem).wait()

    issue(0, csem0)

    def chunk_body(jj, carry, bb=0):
        j = bb * 16 + jj
        kh, ph, kt, pt, oh0, oh1, ot0, ot1 = carry
        jp = j % 2
        col = wcol0 + j * _CHC
        valid = (j < _NCH) & (col < _TCOLS)
        nxt = (j + 1 < _NCH) & (col + _CHC < _TCOLS)

        @pl.when(nxt & (jp == 0))
        def _():
            issue(j + 1, csem1)

        @pl.when(nxt & (jp == 1))
        def _():
            issue(j + 1, csem0)

        @pl.when(valid & (jp == 0))
        def _():
            drain_chunk(csem0)

        @pl.when(valid & (jp == 1))
        def _():
            drain_chunk(csem1)

        jp_s = _splat(jp)
        s0 = col_start(j)
        blo = (col * 128 - wlo) * 16384
        bhi = blo + (128 * _CHC) * 16384

        def side_scan(side, base, cnt_bb, scar):
            lst = hlist if side == "h" else tlist
            ngrp = (cnt_bb + 15) // 16

            def extract_one(vals, st):
                m_, k, p, o0, o1 = st
                li = plsc.all_reduce_ffs(m_)
                v = vals.at[li].get(mode="promise_in_bounds")
                m2 = m_ & (lane != li)
                b_s = v & 16383
                lid = v >> 14
                l_s = lid + wlo - s0
                rre, rim = rows_ref[side]
                for c0 in range(4):
                    gre = plsc.load_gather(chk_re, [jp_s, cvecs[c0], l_s])
                    gim = plsc.load_gather(chk_im, [jp_s, cvecs[c0], l_s])
                    rre[p, k, pl.ds(c0 * 16, 16)] = gre
                    rim[p, k, pl.ds(c0 * 16, 16)] = gim
                gg = k // 16
                bref = bidx_ref[side]
                cur = bref[p, pl.ds(gg * 16, 16)]
                bref[p, pl.ds(gg * 16, 16)] = jnp.where(
                    lane == (k - gg * 16), b_s, cur)
                k = k + 1
                full = k >= _GRP

                @pl.when(full & (p == 0))
                def _():
                    fire(side, 0)

                    @pl.when(o1 > 0)
                    def _():
                        drain_scatter(side, 1)

                @pl.when(full & (p == 1))
                def _():
                    fire(side, 1)

                    @pl.when(o0 > 0)
                    def _():
                        drain_scatter(side, 0)

                o0 = jnp.where(full & (p == 0), 1, jnp.where(full, 0, o0))
                o1 = jnp.where(full & (p == 1), 1, jnp.where(full, 0, o1))
                p = jnp.where(full, 1 - p, p)
                k = jnp.where(full, 0, k)
                return m2, k, p, o0, o1

            def grp_body(g, c2):
                vals = lst[pl.ds(base + g * 16, 16)]
                m0 = (vals >= blo) & (vals < bhi)
                res = lax.while_loop(
                    lambda s: jnp.any(s[0]),
                    lambda s, _v=vals: extract_one(_v, s),
                    (m0,) + c2)
                return res[1:]

            return lax.fori_loop(0, ngrp, grp_body, scar)

        kh, ph, oh0, oh1 = side_scan(
            "h", hoffs[bb], hcs[bb], (kh, ph, oh0, oh1))
        kt, pt, ot0, ot1 = side_scan(
            "t", toffs[bb], tcs[bb], (kt, pt, ot0, ot1))
        return kh, ph, kt, pt, oh0, oh1, ot0, ot1

    carry = (zero, zero, zero, zero, zero, zero, zero, zero)
    for _bb in range(_NSUB):
        carry = lax.fori_loop(
            0, 16, functools.partial(chunk_body, bb=_bb), carry)
    kh, ph, kt, pt, oh0, oh1, ot0, ot1 = carry

    @pl.when((kh > 0) & (ph == 0))
    def _():
        fire("h", 0)

    @pl.when((kh > 0) & (ph == 1))
    def _():
        fire("h", 1)

    @pl.when((kt > 0) & (pt == 0))
    def _():
        fire("t", 0)

    @pl.when((kt > 0) & (pt == 1))
    def _():
        fire("t", 1)

    oh0 = jnp.where((kh > 0) & (ph == 0), 1, oh0)
    oh1 = jnp.where((kh > 0) & (ph == 1), 1, oh1)
    ot0 = jnp.where((kt > 0) & (pt == 0), 1, ot0)
    ot1 = jnp.where((kt > 0) & (pt == 1), 1, ot1)

    @pl.when(oh0 > 0)
    def _():
        drain_scatter("h", 0)

    @pl.when(oh1 > 0)
    def _():
        drain_scatter("h", 1)

    @pl.when(ot0 > 0)
    def _():
        drain_scatter("t", 0)

    @pl.when(ot1 > 0)
    def _():
        drain_scatter("t", 1)


@functools.partial(
    pl.kernel,
    mesh=_mesh,
    out_type=jax.ShapeDtypeStruct((B,), jnp.float32),
    compiler_params=_params,
    scratch_types=[
        pltpu.VMEM((2, 64, 128), jnp.float32),
        pltpu.VMEM((2, 64, 128), jnp.float32),
        pltpu.VMEM((2, 64, 128), jnp.float32),
        pltpu.VMEM((2, 64, 128), jnp.float32),
        pltpu.VMEM((2, 64, 128), jnp.float32),
        pltpu.VMEM((2, 64, 128), jnp.float32),
        pltpu.VMEM((2, 64), jnp.int32),
        pltpu.VMEM((64,), jnp.float32),
        pltpu.SemaphoreType.DMA,
        pltpu.SemaphoreType.DMA,
    ],
)
def _score_kernel(hre, him, tre, tim, rel_re, rel_im, rels, out,
                  bh_re, bh_im, bt_re, bt_im, brr, bri, ridx, outv,
                  sem0, sem1):
    lane = lax.iota(jnp.int32, 16)
    wid = lax.axis_index("s") * _NC + lax.axis_index("c")
    wb = wid * (B // _NW)
    bufs = (bh_re, bh_im, bt_re, bt_im, brr, bri)

    def issue_sub(sc, p, sem):
        base = wb + sc * 64
        pltpu.sync_copy(rels.at[pl.ds(base, 64)], ridx.at[p])
        pltpu.async_copy(hre.at[pl.ds(base, 64), :], bh_re.at[p], sem)
        pltpu.async_copy(him.at[pl.ds(base, 64), :], bh_im.at[p], sem)
        pltpu.async_copy(tre.at[pl.ds(base, 64), :], bt_re.at[p], sem)
        pltpu.async_copy(tim.at[pl.ds(base, 64), :], bt_im.at[p], sem)
        pltpu.async_copy(rel_re.at[ridx.at[p]], brr.at[p], sem)
        pltpu.async_copy(rel_im.at[ridx.at[p]], bri.at[p], sem)

    def drain_sub(p, sem):
        for ref in bufs:
            pltpu.make_async_copy(
                hre.at[pl.ds(0, 64), :], ref.at[p], sem).wait()

    issue_sub(0, 0, sem0)

    def sub_body(sc, carry):
        base = wb + sc * 64
        jp = sc % 2
        nxt = sc + 1 < 8

        @pl.when(nxt & (jp == 0))
        def _():
            issue_sub(sc + 1, 1, sem1)

        @pl.when(nxt & (jp == 1))
        def _():
            issue_sub(sc + 1, 0, sem0)

        @pl.when(jp == 0)
        def _():
            drain_sub(0, sem0)

        @pl.when(jp == 1)
        def _():
            drain_sub(1, sem1)

        def group(g, c2):
            out16 = jnp.zeros((16,), jnp.float32)
            for jj in range(16):
                row = g * 16 + jj
                acc = None
                for c0 in range(4):
                    sl = pl.ds(c0 * 16, 16)
                    hr = bh_re[jp, row, sl]
                    hi = bh_im[jp, row, sl]
                    tr = bt_re[jp, row, sl]
                    ti = bt_im[jp, row, sl]
                    rr = brr[jp, row, sl]
                    ri = bri[jp, row, sl]
                    term = rr * (hr * tr + hi * ti) + ri * (hr * ti - hi * tr)
                    acc = term if acc is None else acc + term
                s = lax.reduce_sum(acc, axes=(0,))
                out16 = jnp.where(lane == jj, s, out16)
            outv[pl.ds(g * 16, 16)] = out16
            return c2

        lax.fori_loop(0, 4, group, 0)
        pltpu.sync_copy(outv, out.at[pl.ds(base, 64)])
        return carry

    lax.fori_loop(0, 8, sub_body, 0)


def kernel(heads, rels, tails, ent_re, ent_im, rel_re, rel_im):
    heads = heads.astype(jnp.int32)
    rels = rels.astype(jnp.int32)
    tails = tails.astype(jnp.int32)
    hre, him, tre, tim = _sweep_kernel(ent_re.T, ent_im.T, heads, tails)
    rel_re128 = jnp.pad(rel_re, ((0, 0), (0, 64)))
    rel_im128 = jnp.pad(rel_im, ((0, 0), (0, 64)))
    score = _score_kernel(hre, him, tre, tim, rel_re128, rel_im128, rels)
    return score - _TAU


# speedup vs baseline: 1.0564x; 1.0018x over previous
import functools

import jax
import jax.numpy as jnp
from jax import lax
from jax.experimental import pallas as pl
from jax.experimental.pallas import tpu as pltpu
from jax.experimental.pallas import tpu_sc as plsc

B = 16384
DIM = 64
N_ENT = 1000000
N_REL = 1000
_TAU = 0.0

_info = plsc.get_sparse_core_info()
_NC = _info.num_cores
_NS = _info.num_subcores
_NW = _NC * _NS
_TCOLS = (N_ENT + 127) // 128
_CPW = 245
_CHC = 1
_NCH = 245
_SB = B + 8
_GRP = 32

_mesh = plsc.VectorSubcoreMesh(core_axis_name="c", subcore_axis_name="s")
_params = pltpu.CompilerParams(needs_layout_passes=False)


def _splat(x):
    return jnp.zeros((16,), jnp.int32) + x


@functools.partial(
    pl.kernel,
    mesh=_mesh,
    out_type=(
        jax.ShapeDtypeStruct((_SB, 128), jnp.float32),
        jax.ShapeDtypeStruct((_SB, 128), jnp.float32),
        jax.ShapeDtypeStruct((_SB, 128), jnp.float32),
        jax.ShapeDtypeStruct((_SB, 128), jnp.float32),
    ),
    compiler_params=_params,
    scratch_types=[
        pltpu.VMEM((2, 64, 128 * _CHC), jnp.float32),
        pltpu.VMEM((2, 64, 128 * _CHC), jnp.float32),
        pltpu.VMEM((2048,), jnp.int32),
        pltpu.VMEM((2048,), jnp.int32),
        pltpu.VMEM((B + 272,), jnp.int32),
        pltpu.VMEM((B + 272,), jnp.int32),
        pltpu.VMEM((B + 16,), jnp.int32),
        pltpu.VMEM((2, _GRP, 128), jnp.float32),
        pltpu.VMEM((2, _GRP, 128), jnp.float32),
        pltpu.VMEM((2, _GRP, 128), jnp.float32),
        pltpu.VMEM((2, _GRP, 128), jnp.float32),
        pltpu.VMEM((2, _GRP), jnp.int32),
        pltpu.VMEM((2, _GRP), jnp.int32),
        pltpu.SemaphoreType.DMA,
        pltpu.SemaphoreType.DMA,
        pltpu.SemaphoreType.DMA,
        pltpu.SemaphoreType.DMA,
        pltpu.SemaphoreType.DMA,
        pltpu.SemaphoreType.DMA,
    ],
)
def _sweep_kernel(entT_re, entT_im, heads, tails,
                  hre, him, tre, tim,
                  chk_re, chk_im, hblk, tblk, hlist, tlist, tmpl,
                  hrow_re, hrow_im, trow_re, trow_im, hbidx, tbidx,
                  csem0, csem1, hsem0, hsem1, tsem0, tsem1):
    lane = lax.iota(jnp.int32, 16)
    wid = lax.axis_index("s") * _NC + lax.axis_index("c")
    wcol0 = wid * _CPW
    wlo = wcol0 * 128
    whi = jnp.minimum(wlo + _CPW * 128, N_ENT)

    rows_ref = {"h": (hrow_re, hrow_im), "t": (trow_re, trow_im)}
    bidx_ref = {"h": hbidx, "t": tbidx}
    outs_ref = {"h": (hre, him), "t": (tre, tim)}
    sems = {"h": (hsem0, hsem1), "t": (tsem0, tsem1)}

    for side in ("h", "t"):
        for p in (0, 1):
            for q in range(_GRP // 16):
                bidx_ref[side][p, pl.ds(q * 16, 16)] = _splat(B)

    zero = jnp.zeros((), jnp.int32)
    _NSUB = 16

    def build_side(src, blk, lst):
        def bblk(blk_i, cnt):
            pltpu.sync_copy(src.at[pl.ds(blk_i * 2048, 2048)], blk)

            def sg(g, c):
                ids = blk[pl.ds(g * 16, 16)]
                b_vec = blk_i * 2048 + g * 16 + lane
                m = (ids >= wlo) & (ids < whi)
                plsc.store_compressed(tmpl.at[pl.ds(c, 16)],
                                      (ids - wlo) * 16384 + b_vec, mask=m)
                return c + plsc.all_reduce_population_count(m)[0]

            return lax.fori_loop(0, 128, sg, cnt)

        cnt = lax.fori_loop(0, 8, bblk, zero)
        ngrp = (cnt + 15) // 16

        def hist(g, cs):
            vals = tmpl[pl.ds(g * 16, 16)]
            mt = lane < (cnt - g * 16)
            bk = vals >> 25
            return tuple(
                cs[bb] + plsc.all_reduce_population_count(
                    mt & (bk == bb))[0] for bb in range(_NSUB))

        cs = lax.fori_loop(0, ngrp, hist, (zero,) * _NSUB)
        offs = [zero]
        for bb in range(_NSUB):
            offs.append((offs[bb] + cs[bb] + 15) // 16 * 16)

        def fill(i, c):
            lst[pl.ds(i * 16, 16)] = _splat(0x7FFFFFFF)
            return c

        lax.fori_loop(0, (B + 272) // 16, fill, zero)

        def place(g, ptrs):
            vals = tmpl[pl.ds(g * 16, 16)]
            mt = lane < (cnt - g * 16)
            bk = vals >> 25
            nptrs = []
            for bb in range(_NSUB):
                mb = mt & (bk == bb)
                plsc.store_compressed(lst.at[pl.ds(ptrs[bb], 16)], vals,
                                      mask=mb)
                nptrs.append(
                    ptrs[bb] + plsc.all_reduce_population_count(mb)[0])
            return tuple(nptrs)

        lax.fori_loop(0, ngrp, place, tuple(offs[:_NSUB]))
        return offs[:_NSUB], cs

    hoffs, hcs = build_side(heads, hblk, hlist)
    toffs, tcs = build_side(tails, tblk, tlist)

    cvecs = [lax.iota(jnp.int32, 16) + c0 * 16 for c0 in range(4)]

    _W = 128 * _CHC

    def col_start(j):
        return jnp.minimum(wcol0 + j * _CHC, _TCOLS - _CHC) * 128

    def issue(j, sem):
        s0 = col_start(j)
        cp1 = pltpu.async_copy(
            entT_re.at[:, pl.ds(s0, _W)], chk_re.at[j % 2], sem)
        cp2 = pltpu.async_copy(
            entT_im.at[:, pl.ds(s0, _W)], chk_im.at[j % 2], sem)
        return cp1, cp2

    def drain_chunk(sem):
        pltpu.make_async_copy(
            entT_re.at[:, pl.ds(0, _W)], chk_re.at[0], sem).wait()
        pltpu.make_async_copy(
            entT_im.at[:, pl.ds(0, _W)], chk_im.at[0], sem).wait()

    def fire(side, p):
        rre, rim = rows_ref[side]
        ore, oim = outs_ref[side]
        sem = sems[side][p]
        idx = bidx_ref[side].at[p]
        pltpu.async_copy(rre.at[p], ore.at[idx], sem)
        pltpu.async_copy(rim.at[p], oim.at[idx], sem)

    def drain_scatter(side, p):
        rre, rim = rows_ref[side]
        sem = sems[side][p]
        pltpu.make_async_copy(
            entT_re.at[pl.ds(0, _GRP), pl.ds(0, 128)], rre.at[p], sem).wait()
        pltpu.make_async_copy(
            entT_re.at[pl.ds(0, _GRP), pl.ds(0, 128)], rim.at[p], sem).wait()

    issue(0, csem0)

    def chunk_body(jj, carry, bb=0):
        j = bb * 16 + jj
        kh, ph, kt, pt, oh0, oh1, ot0, ot1 = carry
        jp = j % 2
        col = wcol0 + j * _CHC
        valid = (j < _NCH) & (col < _TCOLS)
        nxt = (j + 1 < _NCH) & (col + _CHC < _TCOLS)

        @pl.when(nxt & (jp == 0))
        def _():
            issue(j + 1, csem1)

        @pl.when(nxt & (jp == 1))
        def _():
            issue(j + 1, csem0)

        @pl.when(valid & (jp == 0))
        def _():
            drain_chunk(csem0)

        @pl.when(valid & (jp == 1))
        def _():
            drain_chunk(csem1)

        jp_s = _splat(jp)
        s0 = col_start(j)
        blo = (col * 128 - wlo) * 16384
        bhi = blo + (128 * _CHC) * 16384

        def side_scan(side, base, cnt_bb, scar):
            lst = hlist if side == "h" else tlist
            ngrp = (cnt_bb + 15) // 16

            def extract_one(vals, st):
                m_, k, p, o0, o1 = st
                li = plsc.all_reduce_ffs(m_)
                v = vals.at[li].get(mode="promise_in_bounds")
                m2 = m_ & (lane != li)
                b_s = v & 16383
                lid = v >> 14
                l_s = lid + wlo - s0
                rre, rim = rows_ref[side]
                for c0 in range(4):
                    gre = plsc.load_gather(chk_re, [jp_s, cvecs[c0], l_s])
                    gim = plsc.load_gather(chk_im, [jp_s, cvecs[c0], l_s])
                    rre[p, k, pl.ds(c0 * 16, 16)] = gre
                    rim[p, k, pl.ds(c0 * 16, 16)] = gim
                gg = k // 16
                bref = bidx_ref[side]
                cur = bref[p, pl.ds(gg * 16, 16)]
                bref[p, pl.ds(gg * 16, 16)] = jnp.where(
                    lane == (k - gg * 16), b_s, cur)
                k = k + 1
                full = k >= _GRP

                @pl.when(full & (p == 0))
                def _():
                    fire(side, 0)

                    @pl.when(o1 > 0)
                    def _():
                        drain_scatter(side, 1)

                @pl.when(full & (p == 1))
                def _():
                    fire(side, 1)

                    @pl.when(o0 > 0)
                    def _():
                        drain_scatter(side, 0)

                o0 = jnp.where(full & (p == 0), 1, jnp.where(full, 0, o0))
                o1 = jnp.where(full & (p == 1), 1, jnp.where(full, 0, o1))
                p = jnp.where(full, 1 - p, p)
                k = jnp.where(full, 0, k)
                return m2, k, p, o0, o1

            def grp_body(g, c2):
                vals = lst[pl.ds(base + g * 16, 16)]
                m0 = (vals >= blo) & (vals < bhi)
                res = lax.while_loop(
                    lambda s: jnp.any(s[0]),
                    lambda s, _v=vals: extract_one(_v, s),
                    (m0,) + c2)
                return res[1:]

            return lax.fori_loop(0, ngrp, grp_body, scar)

        kh, ph, oh0, oh1 = side_scan(
            "h", hoffs[bb], hcs[bb], (kh, ph, oh0, oh1))
        kt, pt, ot0, ot1 = side_scan(
            "t", toffs[bb], tcs[bb], (kt, pt, ot0, ot1))
        return kh, ph, kt, pt, oh0, oh1, ot0, ot1

    carry = (zero, zero, zero, zero, zero, zero, zero, zero)
    for _bb in range(_NSUB):
        carry = lax.fori_loop(
            0, 16, functools.partial(chunk_body, bb=_bb), carry)
    kh, ph, kt, pt, oh0, oh1, ot0, ot1 = carry

    @pl.when((kh > 0) & (ph == 0))
    def _():
        fire("h", 0)

    @pl.when((kh > 0) & (ph == 1))
    def _():
        fire("h", 1)

    @pl.when((kt > 0) & (pt == 0))
    def _():
        fire("t", 0)

    @pl.when((kt > 0) & (pt == 1))
    def _():
        fire("t", 1)

    oh0 = jnp.where((kh > 0) & (ph == 0), 1, oh0)
    oh1 = jnp.where((kh > 0) & (ph == 1), 1, oh1)
    ot0 = jnp.where((kt > 0) & (pt == 0), 1, ot0)
    ot1 = jnp.where((kt > 0) & (pt == 1), 1, ot1)

    @pl.when(oh0 > 0)
    def _():
        drain_scatter("h", 0)

    @pl.when(oh1 > 0)
    def _():
        drain_scatter("h", 1)

    @pl.when(ot0 > 0)
    def _():
        drain_scatter("t", 0)

    @pl.when(ot1 > 0)
    def _():
        drain_scatter("t", 1)


@functools.partial(
    pl.kernel,
    mesh=_mesh,
    out_type=jax.ShapeDtypeStruct((B,), jnp.float32),
    compiler_params=_params,
    scratch_types=[
        pltpu.VMEM((2, 64, 128), jnp.float32),
        pltpu.VMEM((2, 64, 128), jnp.float32),
        pltpu.VMEM((2, 64, 128), jnp.float32),
        pltpu.VMEM((2, 64, 128), jnp.float32),
        pltpu.VMEM((2, 64, 128), jnp.float32),
        pltpu.VMEM((2, 64, 128), jnp.float32),
        pltpu.VMEM((2, 64), jnp.int32),
        pltpu.VMEM((64,), jnp.float32),
        pltpu.SemaphoreType.DMA,
        pltpu.SemaphoreType.DMA,
    ],
)
def _score_kernel(hre, him, tre, tim, rel_re, rel_im, rels, out,
                  bh_re, bh_im, bt_re, bt_im, brr, bri, ridx, outv,
                  sem0, sem1):
    lane = lax.iota(jnp.int32, 16)
    wid = lax.axis_index("s") * _NC + lax.axis_index("c")
    wb = wid * (B // _NW)
    bufs = (bh_re, bh_im, bt_re, bt_im, brr, bri)

    def issue_sub(sc, p, sem):
        base = wb + sc * 64
        pltpu.sync_copy(rels.at[pl.ds(base, 64)], ridx.at[p])
        pltpu.async_copy(hre.at[pl.ds(base, 64), :], bh_re.at[p], sem)
        pltpu.async_copy(him.at[pl.ds(base, 64), :], bh_im.at[p], sem)
        pltpu.async_copy(tre.at[pl.ds(base, 64), :], bt_re.at[p], sem)
        pltpu.async_copy(tim.at[pl.ds(base, 64), :], bt_im.at[p], sem)
        pltpu.async_copy(rel_re.at[ridx.at[p]], brr.at[p], sem)
        pltpu.async_copy(rel_im.at[ridx.at[p]], bri.at[p], sem)

    def drain_sub(p, sem):
        for ref in bufs:
            pltpu.make_async_copy(
                hre.at[pl.ds(0, 64), :], ref.at[p], sem).wait()

    issue_sub(0, 0, sem0)

    def sub_body(sc, carry):
        base = wb + sc * 64
        jp = sc % 2
        nxt = sc + 1 < 8

        @pl.when(nxt & (jp == 0))
        def _():
            issue_sub(sc + 1, 1, sem1)

        @pl.when(nxt & (jp == 1))
        def _():
            issue_sub(sc + 1, 0, sem0)

        @pl.when(jp == 0)
        def _():
            drain_sub(0, sem0)

        @pl.when(jp == 1)
        def _():
            drain_sub(1, sem1)

        def group(g, c2):
            out16 = jnp.zeros((16,), jnp.float32)
            for jj in range(16):
                row = g * 16 + jj
                acc = None
                for c0 in range(4):
                    sl = pl.ds(c0 * 16, 16)
                    hr = bh_re[jp, row, sl]
                    hi = bh_im[jp, row, sl]
                    tr = bt_re[jp, row, sl]
                    ti = bt_im[jp, row, sl]
                    rr = brr[jp, row, sl]
                    ri = bri[jp, row, sl]
                    term = rr * (hr * tr + hi * ti) + ri * (hr * ti - hi * tr)
                    acc = term if acc is None else acc + term
                s = lax.reduce_sum(acc, axes=(0,))
                out16 = jnp.where(lane == jj, s, out16)
            outv[pl.ds(g * 16, 16)] = out16
            return c2

        lax.fori_loop(0, 4, group, 0)
        pltpu.sync_copy(outv, out.at[pl.ds(base, 64)])
        return carry

    lax.fori_loop(0, 8, sub_body, 0)


def kernel(heads, rels, tails, ent_re, ent_im, rel_re, rel_im):
    heads = heads.astype(jnp.int32)
    rels = rels.astype(jnp.int32)
    tails = tails.astype(jnp.int32)
    hre, him, tre, tim = _sweep_kernel(ent_re.T, ent_im.T, heads, tails)
    rel_re128 = jnp.pad(rel_re, ((0, 0), (0, 64)))
    rel_im128 = jnp.pad(rel_im, ((0, 0), (0, 64)))
    score = _score_kernel(hre, him, tre, tim, rel_re128, rel_im128, rels)
    return score - _TAU
